# baseline (speedup 1.0000x reference)
import jax
import jax.numpy as jnp
from jax import lax
from jax.experimental import pallas as pl
from jax.experimental.pallas import tpu as pltpu
from jax.experimental.pallas import tpu_sc as plsc

N_FIELDS = 26
VOCAB = 100000
DIM = 32
BATCH = 16384

ROWS = BATCH * N_FIELDS
FD = N_FIELDS * DIM
NC, NS, LANES = 2, 16, 16
NW = NC * NS
B_PER_W = BATCH // NW
ROWS_PER_W = ROWS // NW
EX = LANES
CHUNK = EX * N_FIELDS
BLK = CHUNK // 4
GPC = CHUNK // BLK
NCHUNK = B_PER_W // EX
OFF_PERIOD = 1664
OFF_VREGS = OFF_PERIOD // LANES


def _sc_body(tab_hbm, idx_hbm, offs_hbm, out_hbm,
             idx_v, offs_v, rows_a, rows_b, outt_a, outt_b,
             sem_g0, sem_g1, sem_w0, sem_w1):
    sem_g = (sem_g0, sem_g1)
    sem_w = (sem_w0, sem_w1)
    rows = (rows_a, rows_b)
    outts = (outt_a, outt_b)
    wid = lax.axis_index("s") * NC + lax.axis_index("c")
    base = wid * ROWS_PER_W
    b0 = wid * B_PER_W

    pltpu.sync_copy(idx_hbm.at[pl.ds(base, ROWS_PER_W)], idx_v)
    pltpu.sync_copy(offs_hbm, offs_v)

    def prep(i, carry):
        sl = pl.ds(i * LANES, LANES)
        ph = pl.ds((i % OFF_VREGS) * LANES, LANES)
        idx_v[sl] = idx_v[sl] + offs_v[ph]
        return carry

    lax.fori_loop(0, ROWS_PER_W // LANES, prep, 0)

    def fire_gathers(c, par):
        for j in range(GPC):
            idx_slice = idx_v.at[pl.ds(c * CHUNK + j * BLK, BLK)]
            pltpu.async_copy(tab_hbm.at[idx_slice],
                             rows[par].at[pl.ds(j * BLK, BLK)], sem_g[par])

    def wait_gathers(par):
        for j in range(GPC):
            pltpu.make_async_copy(tab_hbm.at[idx_v.at[pl.ds(0, BLK)]],
                                  rows[par].at[pl.ds(0, BLK)],
                                  sem_g[par]).wait()

    def wait_write(par):
        pltpu.make_async_copy(outts[par], out_hbm.at[:, pl.ds(0, EX)],
                              sem_w[par]).wait()

    iota = lax.iota(jnp.int32, LANES)
    zeros = jnp.zeros((LANES,), jnp.int32)
    ex_rows = iota * N_FIELDS
    dcol = [jnp.full((LANES,), d, jnp.int32) for d in range(DIM)]

    def transpose(par):
        rb, ob = rows[par], outts[par]

        def f_body(f, carry):
            jvec = ex_rows + f
            for d in range(DIM):
                val = plsc.load_gather(rb, [jvec, dcol[d]])
                plsc.store_scatter(ob, [zeros + (f * DIM + d), iota], val)
            return carry

        lax.fori_loop(0, N_FIELDS, f_body, 0)

    def fire_write(c, par):
        pltpu.async_copy(outts[par],
                         out_hbm.at[:, pl.ds(b0 + c * EX, EX)], sem_w[par])

    fire_gathers(0, 0)
    fire_gathers(1, 1)

    def pair_body(g, carry):
        c0 = g * 2
        for par in range(2):
            c = c0 + par
            wait_gathers(par)

            @pl.when(c >= 2)
            def _():
                wait_write(par)

            transpose(par)
            fire_write(c, par)

            @pl.when(c + 2 < NCHUNK)
            def _():
                fire_gathers(c + 2, par)
        return carry

    lax.fori_loop(0, NCHUNK // 2, pair_body, 0)
    wait_write(0)
    wait_write(1)


@jax.jit
def kernel(inputs, tables):
    flat_tables = tables.reshape(N_FIELDS * VOCAB, DIM)
    idx_flat = inputs.reshape(ROWS)
    offs = (jnp.arange(OFF_PERIOD, dtype=jnp.int32) % N_FIELDS) * VOCAB

    mesh = plsc.VectorSubcoreMesh(core_axis_name="c", subcore_axis_name="s")
    out_t = pl.kernel(
        _sc_body,
        out_type=jax.ShapeDtypeStruct((FD, BATCH), jnp.float32),
        mesh=mesh,
        compiler_params=pltpu.CompilerParams(use_tc_tiling_on_sc=False,
                                             needs_layout_passes=False),
        scratch_types=[
            pltpu.VMEM((ROWS_PER_W,), jnp.int32),
            pltpu.VMEM((OFF_PERIOD,), jnp.int32),
            pltpu.VMEM((CHUNK, DIM), jnp.float32),
            pltpu.VMEM((CHUNK, DIM), jnp.float32),
            pltpu.VMEM((FD, EX), jnp.float32),
            pltpu.VMEM((FD, EX), jnp.float32),
            pltpu.SemaphoreType.DMA,
            pltpu.SemaphoreType.DMA,
            pltpu.SemaphoreType.DMA,
            pltpu.SemaphoreType.DMA,
        ],
    )(flat_tables, idx_flat, offs)
    return jnp.swapaxes(out_t, 0, 1)


# speedup vs baseline: 1.1843x; 1.1843x over previous
import jax
import jax.numpy as jnp
from jax import lax
from jax.experimental import pallas as pl
from jax.experimental.pallas import tpu as pltpu
from jax.experimental.pallas import tpu_sc as plsc

N_FIELDS = 26
VOCAB = 100000
DIM = 32
BATCH = 16384

ROWS = BATCH * N_FIELDS
NC, NS, LANES = 2, 16, 16
NW = NC * NS
ROWS_PER_W = ROWS // NW
BLK = 128
OFF_PERIOD = 1664
OFF_VREGS = OFF_PERIOD // LANES
CHUNK = 1024
GPC = CHUNK // BLK
NCHUNK = ROWS_PER_W // CHUNK


def _sc_body(tab_hbm, idx_hbm, offs_hbm, out_hbm,
             idx_v, offs_v, rows_a, rows_b, sem_g0, sem_g1, sem_w0, sem_w1):
    sem_g = (sem_g0, sem_g1)
    sem_w = (sem_w0, sem_w1)
    wid = lax.axis_index("s") * NC + lax.axis_index("c")
    base = wid * ROWS_PER_W

    pltpu.sync_copy(idx_hbm.at[pl.ds(base, ROWS_PER_W)], idx_v)
    pltpu.sync_copy(offs_hbm, offs_v)

    bufs = (rows_a, rows_b)

    def add_off_chunk(c):
        v0 = c * (CHUNK // LANES)

        def body(i, carry):
            j = v0 + i
            sl = pl.ds(j * LANES, LANES)
            ph = pl.ds((j % OFF_VREGS) * LANES, LANES)
            idx_v[sl] = idx_v[sl] + offs_v[ph]
            return carry

        lax.fori_loop(0, CHUNK // LANES, body, 0)

    def fire_gathers(c):
        buf = bufs[c % 2]
        descs = []
        for j in range(GPC):
            idx_slice = idx_v.at[pl.ds(c * CHUNK + j * BLK, BLK)]
            descs.append(
                pltpu.async_copy(tab_hbm.at[idx_slice],
                                 buf.at[pl.ds(j * BLK, BLK)], sem_g[c % 2]))
        return descs

    def fire_write(c):
        return pltpu.async_copy(bufs[c % 2],
                                out_hbm.at[pl.ds(base + c * CHUNK, CHUNK)],
                                sem_w[c % 2])

    add_off_chunk(0)
    gathers = fire_gathers(0)
    writes = [None, None]
    for c in range(NCHUNK):
        if c + 1 < NCHUNK:
            add_off_chunk(c + 1)
            if writes[(c + 1) % 2] is not None:
                writes[(c + 1) % 2].wait()
            nxt = fire_gathers(c + 1)
        for d in gathers:
            d.wait()
        writes[c % 2] = fire_write(c)
        if c + 1 < NCHUNK:
            gathers = nxt
    writes[(NCHUNK - 1) % 2].wait()
    if NCHUNK > 1:
        writes[(NCHUNK - 2) % 2].wait()


@jax.jit
def kernel(inputs, tables):
    flat_tables = tables.reshape(N_FIELDS * VOCAB, DIM)
    idx_flat = inputs.reshape(ROWS)
    offs = (jnp.arange(OFF_PERIOD, dtype=jnp.int32) % N_FIELDS) * VOCAB

    mesh = plsc.VectorSubcoreMesh(core_axis_name="c", subcore_axis_name="s")
    out = pl.kernel(
        _sc_body,
        out_type=jax.ShapeDtypeStruct((ROWS, DIM), jnp.float32),
        mesh=mesh,
        compiler_params=pltpu.CompilerParams(use_tc_tiling_on_sc=False),
        scratch_types=[
            pltpu.VMEM((ROWS_PER_W,), jnp.int32),
            pltpu.VMEM((OFF_PERIOD,), jnp.int32),
            pltpu.VMEM((CHUNK, DIM), jnp.float32),
            pltpu.VMEM((CHUNK, DIM), jnp.float32),
            pltpu.SemaphoreType.DMA,
            pltpu.SemaphoreType.DMA,
            pltpu.SemaphoreType.DMA,
            pltpu.SemaphoreType.DMA,
        ],
    )(flat_tables, idx_flat, offs)
    return out.reshape(BATCH, N_FIELDS * DIM)
